# baseline (speedup 1.0000x reference)
import functools

import jax
import jax.numpy as jnp
from jax import lax
from jax.experimental import pallas as pl
from jax.experimental.pallas import tpu as pltpu
from jax.experimental.pallas import tpu_sc as plsc

_NUM_CORES = 2
_NUM_SUBCORES = 16
_NW = _NUM_CORES * _NUM_SUBCORES

_NBUF = 4
_CHUNK = 416


def _make_gather(num_rows, dim, batch):
    assert batch % (_NW * _CHUNK) == 0
    b_per_w = batch // _NW
    nchunks = b_per_w // _CHUNK
    assert nchunks >= 2 * _NBUF
    mesh = plsc.VectorSubcoreMesh(core_axis_name="c", subcore_axis_name="s")

    @functools.partial(
        pl.kernel,
        mesh=mesh,
        compiler_params=pltpu.CompilerParams(use_tc_tiling_on_sc=False),
        out_type=jax.ShapeDtypeStruct((batch, dim), jnp.float32),
        scratch_types=[
            pltpu.VMEM((b_per_w,), jnp.int32),
            pltpu.VMEM((_NBUF, _CHUNK, dim), jnp.float32),
            pltpu.SemaphoreType.DMA((_NBUF,)),
            pltpu.SemaphoreType.DMA((_NBUF,)),
        ],
    )
    def gather(table_hbm, idx_hbm, out_hbm, idx_v, rows_v, gsem, ssem):
        wid = lax.axis_index("s") * _NUM_CORES + lax.axis_index("c")
        base = wid * b_per_w
        pltpu.sync_copy(idx_hbm.at[pl.ds(base, b_per_w)], idx_v)

        def start_gather(g, b):
            pltpu.async_copy(
                table_hbm.at[idx_v.at[pl.ds(g * _CHUNK, _CHUNK)]],
                rows_v.at[b],
                gsem.at[b],
            )

        def start_store(g, b):
            pltpu.async_copy(
                rows_v.at[b],
                out_hbm.at[pl.ds(base + g * _CHUNK, _CHUNK)],
                ssem.at[b],
            )

        def wait_gather(g, b):
            pltpu.make_async_copy(
                table_hbm.at[idx_v.at[pl.ds(g * _CHUNK, _CHUNK)]],
                rows_v.at[b],
                gsem.at[b],
            ).wait()

        def wait_store(g, b):
            pltpu.make_async_copy(
                rows_v.at[b],
                out_hbm.at[pl.ds(base + g * _CHUNK, _CHUNK)],
                ssem.at[b],
            ).wait()

        for i in range(_NBUF):
            start_gather(i, i)
            if i == _NBUF - 1:
                wait_gather(0, 0)
                start_store(0, 0)

        @pl.loop(_NBUF, nchunks)
        def _(i):
            b = lax.rem(i, _NBUF)
            bp = lax.rem(i + 1, _NBUF)
            wait_store(i - _NBUF, b)
            start_gather(i, b)
            wait_gather(i - _NBUF + 1, bp)
            start_store(i - _NBUF + 1, bp)

        for i in range(nchunks, nchunks + _NBUF):
            b = i % _NBUF
            bp = (i + 1) % _NBUF
            wait_store(i - _NBUF, b)
            if i < nchunks + _NBUF - 1:
                wait_gather(i - _NBUF + 1, bp)
                start_store(i - _NBUF + 1, bp)

    return gather


def kernel(indices, weight):
    flat = indices.reshape(-1)
    gather = _make_gather(weight.shape[0], weight.shape[1], flat.shape[0])
    out = gather(weight, flat)
    return out.reshape(indices.shape + (weight.shape[-1],))


# speedup vs baseline: 1.0234x; 1.0234x over previous
import functools

import jax
import jax.numpy as jnp
from jax import lax
from jax.experimental import pallas as pl
from jax.experimental.pallas import tpu as pltpu
from jax.experimental.pallas import tpu_sc as plsc

_NUM_CORES = 2
_NUM_SUBCORES = 16
_NW = _NUM_CORES * _NUM_SUBCORES

_NBUF = 4
_CHUNK = 208


def _make_gather(num_rows, dim, batch, nfields):
    assert batch % (_NW * _CHUNK) == 0
    b_per_w = batch // _NW
    nchunks = b_per_w // _CHUNK
    assert nchunks >= 2 * _NBUF
    pdim = 2 * dim
    mesh = plsc.VectorSubcoreMesh(core_axis_name="c", subcore_axis_name="s")

    @functools.partial(
        pl.kernel,
        mesh=mesh,
        compiler_params=pltpu.CompilerParams(use_tc_tiling_on_sc=False),
        out_type=jax.ShapeDtypeStruct((batch, dim), jnp.float32),
        scratch_types=[
            pltpu.VMEM((b_per_w,), jnp.int32),
            pltpu.VMEM((_NBUF, _CHUNK, pdim), jnp.float32),
            pltpu.SemaphoreType.DMA((_NBUF,)),
            pltpu.SemaphoreType.DMA((_NBUF,)),
        ],
    )
    def gather(table_hbm, idx_hbm, out_hbm, idx_v, rows_v, gsem, ssem):
        wid = lax.axis_index("s") * _NUM_CORES + lax.axis_index("c")
        base = wid * b_per_w
        pltpu.sync_copy(idx_hbm.at[pl.ds(base, b_per_w)], idx_v)
        out_flat = out_hbm

        def start_gather(g, b):
            pltpu.async_copy(
                table_hbm.at[idx_v.at[pl.ds(g * _CHUNK, _CHUNK)]],
                rows_v.at[b],
                gsem.at[b],
            )

        def start_store(g, b):
            pltpu.async_copy(
                rows_v.at[b, :, pl.ds(0, dim)],
                out_flat.at[pl.ds(base + g * _CHUNK, _CHUNK)],
                ssem.at[b],
            )

        def wait_gather(g, b):
            pltpu.make_async_copy(
                table_hbm.at[idx_v.at[pl.ds(g * _CHUNK, _CHUNK)]],
                rows_v.at[b],
                gsem.at[b],
            ).wait()

        def wait_store(g, b):
            pltpu.make_async_copy(
                rows_v.at[b, :, pl.ds(0, dim)],
                out_flat.at[pl.ds(base + g * _CHUNK, _CHUNK)],
                ssem.at[b],
            ).wait()

        for i in range(_NBUF):
            start_gather(i, i)
            if i == _NBUF - 1:
                wait_gather(0, 0)
                start_store(0, 0)

        @pl.loop(_NBUF, nchunks)
        def _(i):
            b = lax.rem(i, _NBUF)
            bp = lax.rem(i + 1, _NBUF)
            wait_store(i - _NBUF, b)
            start_gather(i, b)
            wait_gather(i - _NBUF + 1, bp)
            start_store(i - _NBUF + 1, bp)

        for i in range(nchunks, nchunks + _NBUF):
            b = i % _NBUF
            bp = (i + 1) % _NBUF
            wait_store(i - _NBUF, b)
            if i < nchunks + _NBUF - 1:
                wait_gather(i - _NBUF + 1, bp)
                start_store(i - _NBUF + 1, bp)

    return gather


def kernel(indices, weight):
    num_rows, dim = weight.shape
    flat = indices.reshape(-1)
    wpad = jnp.pad(weight, ((0, 0), (0, dim)))
    gather = _make_gather(num_rows, dim, flat.shape[0], indices.shape[1])
    out = gather(wpad, flat)
    return out.reshape(indices.shape + (dim,))
